# Initial kernel scaffold; baseline (speedup 1.0000x reference)
import jax
import jax.numpy as jnp
from jax.experimental import pallas as pl


def kernel(pixel_values, grid_thw, input_ids, position_ids, attention_mask):
    raise NotImplementedError("write your pallas kernel here")


# speedup vs baseline: 1.0389x; 1.0389x over previous
import functools

import jax
import jax.numpy as jnp
from jax import lax
from jax.experimental import pallas as pl
from jax.experimental.pallas import tpu as pltpu
from jax.experimental.pallas import tpu_sc as plsc

IMAGE_TOKEN_ID = 151655
VIDEO_TOKEN_ID = 151656

BATCH = 4
SEQLEN = 4096
HIDDEN = 1280
T, H, W = 1, 64, 48
N_VIS = T * H * W
CNT = (H // 4) * (W // 4)
N_IMG = (H // 2) * (W // 2)
NKEEP = 100 + CNT + (SEQLEN - 100 - N_IMG)

L = 16
NCH = SEQLEN // L
NSEL = CNT // L
NOUT = NKEEP // L


def _pix_body(i_ref, o_ref):
    o_ref[...] = i_ref[...].reshape(o_ref.shape)


def _pix_compress(pixel_values):
    pv8 = pixel_values.reshape(BATCH, 16, 2, 2, 12, 2, 2, HIDDEN)
    out8 = pl.pallas_call(
        _pix_body,
        grid=(BATCH, 2, 2),
        in_specs=[pl.BlockSpec(
            (1, 16, 1, 2, 12, 1, 2, HIDDEN),
            lambda i, f, p: (i, 0, f, 0, 0, p, 0, 0))],
        out_specs=pl.BlockSpec(
            (1, 1, 16, 2, 12, 1, 2, HIDDEN),
            lambda i, f, p: (i, p, 0, 0, 0, f, 0, 0)),
        out_shape=jax.ShapeDtypeStruct(
            (BATCH, 2, 16, 2, 12, 2, 2, HIDDEN), jnp.float32),
    )(pv8)
    return out8.reshape(BATCH * (H // 2) * (W // 2), 4 * HIDDEN)


def _token_body(ids_hbm, pos_hbm, am_hbm, out_ids, out_pos, out_am,
                ids_v, am_v, pos_v0, pos_v1, pos_v2, mask_v, idx_v,
                ids_o, pos_o0, pos_o1, pos_o2, am_o):
    pos_v = (pos_v0, pos_v1, pos_v2)
    pos_o = (pos_o0, pos_o1, pos_o2)
    wid = lax.axis_index("s") * 2 + lax.axis_index("c")

    @pl.when(wid < BATCH)
    def _():
        i = wid
        pltpu.sync_copy(ids_hbm.at[i], ids_v)
        pltpu.sync_copy(am_hbm.at[i], am_v)
        for k in range(3):
            pltpu.sync_copy(pos_hbm.at[k, i], pos_v[k])

        lane = lax.iota(jnp.int32, L)

        def span_body(c, carry):
            nmn, mx = carry
            t = lane + c * L
            v = ids_v[pl.ds(c * L, L)]
            med = (v == IMAGE_TOKEN_ID) | (v == VIDEO_TOKEN_ID)
            nmn = jnp.maximum(nmn, jnp.where(med, -t, -SEQLEN))
            mx = jnp.maximum(mx, jnp.where(med, t, -1))
            return nmn, mx

        nmn, mx = lax.fori_loop(
            0, NCH, span_body,
            (jnp.full((L,), -SEQLEN, jnp.int32),
             jnp.full((L,), -1, jnp.int32)))
        start = -jnp.max(nmn)
        end = jnp.max(mx)

        def mask_body(c, _):
            t = lane + c * L
            mask_v[pl.ds(c * L, L)] = ((t < start) | (t > end)).astype(jnp.int32)
            return 0

        lax.fori_loop(0, NCH, mask_body, 0)

        startf = start.astype(jnp.float32)
        endf = end.astype(jnp.float32)

        def sel_body(ck, _):
            kf = (lane + ck * L).astype(jnp.float32)
            h = kf / float(CNT - 1)
            sel = (startf * (1.0 - h) + endf * h).astype(jnp.int32)
            plsc.store_scatter(mask_v, [sel], jnp.ones((L,), jnp.int32))
            return 0

        lax.fori_loop(0, NSEL, sel_body, 0)

        def zero_body(j, _):
            idx_v[pl.ds(j * L, L)] = jnp.zeros((L,), jnp.int32)
            return 0

        lax.fori_loop(0, NOUT, zero_body, 0)

        def compact_body(c, base):
            t = lane + c * L
            m = mask_v[pl.ds(c * L, L)]
            a = am_v[pl.ds(c * L, L)]
            m = jnp.where(a != 0, m, 0)
            incl = plsc.cumsum(m)
            dest = base + incl - m
            ok = (m > 0) & (dest < NKEEP)
            plsc.store_scatter(idx_v, [dest], t, mask=ok)
            return base + jnp.sum(m)

        lax.fori_loop(0, NCH, compact_body, jnp.int32(0))

        def gather_body(j, _):
            iv = idx_v[pl.ds(j * L, L)]
            ids_o[pl.ds(j * L, L)] = plsc.load_gather(ids_v, [iv])
            am_o[pl.ds(j * L, L)] = plsc.load_gather(am_v, [iv])
            for k in range(3):
                pos_o[k][pl.ds(j * L, L)] = plsc.load_gather(pos_v[k], [iv])
            return 0

        lax.fori_loop(0, NOUT, gather_body, 0)

        pltpu.sync_copy(ids_o, out_ids.at[i])
        pltpu.sync_copy(am_o, out_am.at[i])
        for k in range(3):
            pltpu.sync_copy(pos_o[k], out_pos.at[k, i])


_token_kernel = functools.partial(
    pl.kernel,
    out_type=[
        jax.ShapeDtypeStruct((BATCH, NKEEP), jnp.int32),
        jax.ShapeDtypeStruct((3, BATCH, NKEEP), jnp.int32),
        jax.ShapeDtypeStruct((BATCH, NKEEP), jnp.int32),
    ],
    mesh=plsc.VectorSubcoreMesh(core_axis_name="c", subcore_axis_name="s"),
    compiler_params=pltpu.CompilerParams(needs_layout_passes=False),
    scratch_types=[
        pltpu.VMEM((SEQLEN,), jnp.int32),
        pltpu.VMEM((SEQLEN,), jnp.int32),
        pltpu.VMEM((SEQLEN,), jnp.int32),
        pltpu.VMEM((SEQLEN,), jnp.int32),
        pltpu.VMEM((SEQLEN,), jnp.int32),
        pltpu.VMEM((SEQLEN,), jnp.int32),
        pltpu.VMEM((NKEEP,), jnp.int32),
        pltpu.VMEM((NKEEP,), jnp.int32),
        pltpu.VMEM((NKEEP,), jnp.int32),
        pltpu.VMEM((NKEEP,), jnp.int32),
        pltpu.VMEM((NKEEP,), jnp.int32),
        pltpu.VMEM((NKEEP,), jnp.int32),
    ],
)(_token_body)


def kernel(pixel_values, grid_thw, input_ids, position_ids, attention_mask):
    pix = _pix_compress(pixel_values)
    ids, pos, am = _token_kernel(input_ids, position_ids, attention_mask)
    thw = jnp.stack(
        [grid_thw[:, 0], grid_thw[:, 1] // 2, grid_thw[:, 2] // 2],
        axis=1).astype(jnp.int32)
    return (pix, thw, ids, pos, am)


# speedup vs baseline: 2.8987x; 2.7902x over previous
import functools

import jax
import jax.numpy as jnp
from jax import lax
from jax.experimental import pallas as pl
from jax.experimental.pallas import tpu as pltpu
from jax.experimental.pallas import tpu_sc as plsc

IMAGE_TOKEN_ID = 151655
VIDEO_TOKEN_ID = 151656

BATCH = 4
SEQLEN = 4096
HIDDEN = 1280
T, H, W = 1, 64, 48
N_VIS = T * H * W
CNT = (H // 4) * (W // 4)
N_IMG = (H // 2) * (W // 2)
NKEEP = 100 + CNT + (SEQLEN - 100 - N_IMG)

L = 16
NCH = SEQLEN // L
NSEL = CNT // L
NOUT = NKEEP // L

NTILES = 32
NCHUNKS = BATCH * N_VIS
PER_TILE = NCHUNKS // NTILES
ROWS = 24
ROUNDS = PER_TILE // ROWS


def _make_perm():
    import numpy as np
    m = np.arange(NCHUNKS)
    i = m // N_VIS
    o = (m // 4) % (N_VIS // 4)
    j = m % 4
    f, e = j // 2, j % 2
    P = o // 384
    A = (o // 24) % 16
    u = (o // 12) % 2
    v = o % 12
    return np.asarray(
        i * N_VIS + (A * 4 + f * 2 + u) * W + (v * 4 + P * 2 + e),
        dtype=np.int32)


_PERM = _make_perm()


def _body(perm_hbm, pv_hbm, ids_hbm, pos_hbm, am_hbm,
          out_pix, out_ids, out_pos, out_am,
          buf0, buf1, idx_all, sem0, sem1,
          ids_v, am_v, pos_v0, pos_v1, pos_v2, mask_v, idx_v,
          ids_o, pos_o0, pos_o1, pos_o2, am_o):
    pos_v = (pos_v0, pos_v1, pos_v2)
    pos_o = (pos_o0, pos_o1, pos_o2)
    bufs = (buf0, buf1)
    sems = (sem0, sem1)
    wid = lax.axis_index("s") * 2 + lax.axis_index("c")
    lane = lax.iota(jnp.int32, L)

    pltpu.sync_copy(perm_hbm.at[pl.ds(wid * PER_TILE, PER_TILE)], idx_all)

    def out_base(r):
        return wid * PER_TILE + r * ROWS

    def gather(r, b):
        idx_r = idx_all.at[pl.ds(r * ROWS, ROWS)]
        return pltpu.async_copy(pv_hbm.at[idx_r], bufs[b], sems[b])

    copies = [gather(0, 0), None]
    for r in range(1, ROUNDS + 1):
        if r < ROUNDS:
            copies[r % 2] = gather(r, r % 2)
        pb = (r - 1) % 2
        copies[pb].wait()
        pltpu.sync_copy(bufs[pb], out_pix.at[pl.ds(out_base(r - 1), ROWS)])

    @pl.when(wid < BATCH)
    def _():
        i = wid
        pltpu.sync_copy(ids_hbm.at[i], ids_v)
        pltpu.sync_copy(am_hbm.at[i], am_v)
        for k in range(3):
            pltpu.sync_copy(pos_hbm.at[k, i], pos_v[k])

        def span_body(c, carry):
            nmn, mx = carry
            t = lane + c * L
            v = ids_v[pl.ds(c * L, L)]
            med = (v == IMAGE_TOKEN_ID) | (v == VIDEO_TOKEN_ID)
            nmn = jnp.maximum(nmn, jnp.where(med, -t, -SEQLEN))
            mx = jnp.maximum(mx, jnp.where(med, t, -1))
            return nmn, mx

        nmn, mx = lax.fori_loop(
            0, NCH, span_body,
            (jnp.full((L,), -SEQLEN, jnp.int32),
             jnp.full((L,), -1, jnp.int32)))
        start = -jnp.max(nmn)
        end = jnp.max(mx)

        def mask_body(c, _):
            t = lane + c * L
            mask_v[pl.ds(c * L, L)] = ((t < start) | (t > end)).astype(jnp.int32)
            return 0

        lax.fori_loop(0, NCH, mask_body, 0)

        startf = start.astype(jnp.float32)
        endf = end.astype(jnp.float32)

        def sel_body(ck, _):
            kf = (lane + ck * L).astype(jnp.float32)
            h = kf / float(CNT - 1)
            sel = (startf * (1.0 - h) + endf * h).astype(jnp.int32)
            plsc.store_scatter(mask_v, [sel], jnp.ones((L,), jnp.int32))
            return 0

        lax.fori_loop(0, NSEL, sel_body, 0)

        def zero_body(j, _):
            idx_v[pl.ds(j * L, L)] = jnp.zeros((L,), jnp.int32)
            return 0

        lax.fori_loop(0, NOUT, zero_body, 0)

        def compact_body(c, base):
            t = lane + c * L
            m = mask_v[pl.ds(c * L, L)]
            a = am_v[pl.ds(c * L, L)]
            m = jnp.where(a != 0, m, 0)
            incl = plsc.cumsum(m)
            dest = base + incl - m
            ok = (m > 0) & (dest < NKEEP)
            plsc.store_scatter(idx_v, [dest], t, mask=ok)
            return base + jnp.sum(m)

        lax.fori_loop(0, NCH, compact_body, jnp.int32(0))

        def gather_body(j, _):
            iv = idx_v[pl.ds(j * L, L)]
            ids_o[pl.ds(j * L, L)] = plsc.load_gather(ids_v, [iv])
            am_o[pl.ds(j * L, L)] = plsc.load_gather(am_v, [iv])
            for k in range(3):
                pos_o[k][pl.ds(j * L, L)] = plsc.load_gather(pos_v[k], [iv])
            return 0

        lax.fori_loop(0, NOUT, gather_body, 0)

        pltpu.sync_copy(ids_o, out_ids.at[i])
        pltpu.sync_copy(am_o, out_am.at[i])
        for k in range(3):
            pltpu.sync_copy(pos_o[k], out_pos.at[k, i])


_sc_kernel = functools.partial(
    pl.kernel,
    out_type=[
        jax.ShapeDtypeStruct((NCHUNKS, HIDDEN), jnp.float32),
        jax.ShapeDtypeStruct((BATCH, NKEEP), jnp.int32),
        jax.ShapeDtypeStruct((3, BATCH, NKEEP), jnp.int32),
        jax.ShapeDtypeStruct((BATCH, NKEEP), jnp.int32),
    ],
    mesh=plsc.VectorSubcoreMesh(core_axis_name="c", subcore_axis_name="s"),
    compiler_params=pltpu.CompilerParams(needs_layout_passes=False),
    scratch_types=[
        pltpu.VMEM((ROWS, HIDDEN), jnp.float32),
        pltpu.VMEM((ROWS, HIDDEN), jnp.float32),
        pltpu.VMEM((PER_TILE,), jnp.int32),
        pltpu.SemaphoreType.DMA,
        pltpu.SemaphoreType.DMA,
        pltpu.VMEM((SEQLEN,), jnp.int32),
        pltpu.VMEM((SEQLEN,), jnp.int32),
        pltpu.VMEM((SEQLEN,), jnp.int32),
        pltpu.VMEM((SEQLEN,), jnp.int32),
        pltpu.VMEM((SEQLEN,), jnp.int32),
        pltpu.VMEM((SEQLEN,), jnp.int32),
        pltpu.VMEM((NKEEP,), jnp.int32),
        pltpu.VMEM((NKEEP,), jnp.int32),
        pltpu.VMEM((NKEEP,), jnp.int32),
        pltpu.VMEM((NKEEP,), jnp.int32),
        pltpu.VMEM((NKEEP,), jnp.int32),
        pltpu.VMEM((NKEEP,), jnp.int32),
    ],
)(_body)


def _thw_body(g_ref, o_ref):
    g = g_ref[...]
    half = g // 2
    col = lax.broadcasted_iota(jnp.int32, g.shape, 1)
    o_ref[...] = jnp.where(col == 0, g, half)


def _thw_compress(grid_thw):
    return pl.pallas_call(
        _thw_body,
        out_shape=jax.ShapeDtypeStruct((BATCH, 3), jnp.int32),
    )(grid_thw)


def kernel(pixel_values, grid_thw, input_ids, position_ids, attention_mask):
    perm = jnp.asarray(_PERM)
    pix2d, ids, pos, am = _sc_kernel(
        perm, pixel_values, input_ids, position_ids, attention_mask)
    pix = pix2d.reshape(BATCH * (H // 2) * (W // 2), 4 * HIDDEN)
    thw = _thw_compress(grid_thw)
    return (pix, thw, ids, pos, am)


# speedup vs baseline: 4.9224x; 1.6982x over previous
import functools

import jax
import jax.numpy as jnp
from jax import lax
from jax.experimental import pallas as pl
from jax.experimental.pallas import tpu as pltpu
from jax.experimental.pallas import tpu_sc as plsc

IMAGE_TOKEN_ID = 151655
VIDEO_TOKEN_ID = 151656

BATCH = 4
SEQLEN = 4096
HIDDEN = 1280
T, H, W = 1, 64, 48
N_VIS = T * H * W
CNT = (H // 4) * (W // 4)
N_IMG = (H // 2) * (W // 2)
NKEEP = 100 + CNT + (SEQLEN - 100 - N_IMG)

L = 16
NCH = SEQLEN // L
NSEL = CNT // L
NOUT = NKEEP // L

NTILES = 32
NCHUNKS = BATCH * N_VIS
OROWS = 8
ROUNDS2 = (NCHUNKS // 4) // (NTILES * OROWS)
IDXSTRIDE = OROWS


def _make_perm():
    import numpy as np
    tbl = np.zeros(4 * NTILES * 128, dtype=np.int32)
    for j in range(4):
        wid = np.arange(NTILES)[:, None, None]
        r = np.arange(ROUNDS2)[None, :, None]
        k = np.arange(IDXSTRIDE)[None, None, :]
        o = wid * (ROUNDS2 * OROWS) + r * OROWS + k
        i = o // 768
        P = (o // 384) % 2
        A = (o // 24) % 16
        u = (o // 12) % 2
        v = o % 12
        f, e = j // 2, j % 2
        perm = (i * N_VIS + (A * 4 + f * 2 + u) * W
                + (v * 4 + P * 2 + e)).reshape(NTILES, ROUNDS2 * IDXSTRIDE)
        sec = np.zeros((NTILES, 128), dtype=np.int32)
        sec[:, :ROUNDS2 * IDXSTRIDE] = perm
        tbl[j * NTILES * 128:(j + 1) * NTILES * 128] = sec.reshape(-1)
    return tbl


_PERM = _make_perm()


def _body(perm_hbm, pv_hbm, ids_hbm, pos_hbm, am_hbm,
          out_pix, out_ids, out_pos, out_am,
          buf0, buf1, ix_all, sem0, sem1, wsem0, wsem1,
          ids_v, am_v, pos_v0, pos_v1, pos_v2, mask_v, idx_v, stage_o):
    pos_v = (pos_v0, pos_v1, pos_v2)
    bufs = (buf0, buf1)
    sems = (sem0, sem1)
    wsems = (wsem0, wsem1)
    wid = lax.axis_index("s") * 2 + lax.axis_index("c")
    lane = lax.iota(jnp.int32, L)

    for j in range(4):
        pltpu.sync_copy(perm_hbm.at[pl.ds(j * NTILES * 128 + wid * 128, 128)],
                        ix_all.at[pl.ds(j * 128, 128)])

    def gather(r, b):
        hs = []
        for j in range(4):
            idx_j = ix_all.at[pl.ds(j * 128 + r * IDXSTRIDE, OROWS)]
            dst = bufs[b].at[:, pl.ds(j * HIDDEN, HIDDEN)]
            hs.append(pltpu.async_copy(pv_hbm.at[idx_j], dst, sems[b]))
        return hs

    def scatter(r, b):
        row0 = wid * (ROUNDS2 * OROWS) + r * OROWS
        return pltpu.async_copy(
            bufs[b], out_pix.at[pl.ds(row0, OROWS)], wsems[b])

    gh = [None, None]
    sh = [None, None]
    for r in range(ROUNDS2):
        b = r % 2
        if sh[b] is not None:
            sh[b].wait()
        gh[b] = gather(r, b)
        if r > 0:
            pb = (r - 1) % 2
            for h in gh[pb]:
                h.wait()
            sh[pb] = scatter(r - 1, pb)
    lb = (ROUNDS2 - 1) % 2
    for h in gh[lb]:
        h.wait()
    sh[lb] = scatter(ROUNDS2 - 1, lb)
    sh[1 - lb].wait()
    sh[lb].wait()

    @pl.when(wid < BATCH)
    def _():
        i = wid
        pltpu.sync_copy(ids_hbm.at[i], ids_v)
        pltpu.sync_copy(am_hbm.at[i], am_v)
        for k in range(3):
            pltpu.sync_copy(pos_hbm.at[k, i], pos_v[k])

        def span_body(c, carry):
            nmn, mx = carry
            t = lane + c * L
            v = ids_v[pl.ds(c * L, L)]
            med = (v == IMAGE_TOKEN_ID) | (v == VIDEO_TOKEN_ID)
            nmn = jnp.maximum(nmn, jnp.where(med, -t, -SEQLEN))
            mx = jnp.maximum(mx, jnp.where(med, t, -1))
            return nmn, mx

        nmn, mx = lax.fori_loop(
            0, NCH, span_body,
            (jnp.full((L,), -SEQLEN, jnp.int32),
             jnp.full((L,), -1, jnp.int32)))
        start = -jnp.max(nmn)
        end = jnp.max(mx)

        def mask_body(c, _):
            t = lane + c * L
            mask_v[pl.ds(c * L, L)] = ((t < start) | (t > end)).astype(jnp.int32)
            return 0

        lax.fori_loop(0, NCH, mask_body, 0)

        startf = start.astype(jnp.float32)
        endf = end.astype(jnp.float32)

        def sel_body(ck, _):
            kf = (lane + ck * L).astype(jnp.float32)
            h = kf / float(CNT - 1)
            sel = (startf * (1.0 - h) + endf * h).astype(jnp.int32)
            plsc.store_scatter(mask_v, [sel], jnp.ones((L,), jnp.int32))
            return 0

        lax.fori_loop(0, NSEL, sel_body, 0)

        def zero_body(j, _):
            idx_v[pl.ds(j * L, L)] = jnp.zeros((L,), jnp.int32)
            return 0

        lax.fori_loop(0, NOUT, zero_body, 0)

        def compact_body(c, base):
            t = lane + c * L
            m = mask_v[pl.ds(c * L, L)]
            a = am_v[pl.ds(c * L, L)]
            m = jnp.where(a != 0, m, 0)
            incl = plsc.cumsum(m)
            dest = base + incl - m
            ok = (m > 0) & (dest < NKEEP)
            plsc.store_scatter(idx_v, [dest], t, mask=ok)
            return base + jnp.sum(m)

        lax.fori_loop(0, NCH, compact_body, jnp.int32(0))

        def make_gather(src_v):
            def gather_body(j, _):
                iv = idx_v[pl.ds(j * L, L)]
                stage_o[pl.ds(j * L, L)] = plsc.load_gather(src_v, [iv])
                return 0
            return gather_body

        lax.fori_loop(0, NOUT, make_gather(ids_v), 0)
        pltpu.sync_copy(stage_o, out_ids.at[i])
        lax.fori_loop(0, NOUT, make_gather(am_v), 0)
        pltpu.sync_copy(stage_o, out_am.at[i])
        for k in range(3):
            lax.fori_loop(0, NOUT, make_gather(pos_v[k]), 0)
            pltpu.sync_copy(stage_o, out_pos.at[k, i])


_sc_kernel = functools.partial(
    pl.kernel,
    out_type=[
        jax.ShapeDtypeStruct((BATCH * N_VIS // 4, 4 * HIDDEN), jnp.float32),
        jax.ShapeDtypeStruct((BATCH, NKEEP), jnp.int32),
        jax.ShapeDtypeStruct((3, BATCH, NKEEP), jnp.int32),
        jax.ShapeDtypeStruct((BATCH, NKEEP), jnp.int32),
    ],
    mesh=plsc.VectorSubcoreMesh(core_axis_name="c", subcore_axis_name="s"),
    compiler_params=pltpu.CompilerParams(needs_layout_passes=False),
    scratch_types=[
        pltpu.VMEM((OROWS, 4 * HIDDEN), jnp.float32),
        pltpu.VMEM((OROWS, 4 * HIDDEN), jnp.float32),
        pltpu.VMEM((4 * 128,), jnp.int32),
        pltpu.SemaphoreType.DMA,
        pltpu.SemaphoreType.DMA,
        pltpu.SemaphoreType.DMA,
        pltpu.SemaphoreType.DMA,
        pltpu.VMEM((SEQLEN,), jnp.int32),
        pltpu.VMEM((SEQLEN,), jnp.int32),
        pltpu.VMEM((SEQLEN,), jnp.int32),
        pltpu.VMEM((SEQLEN,), jnp.int32),
        pltpu.VMEM((SEQLEN,), jnp.int32),
        pltpu.VMEM((SEQLEN,), jnp.int32),
        pltpu.VMEM((NKEEP,), jnp.int32),
        pltpu.VMEM((NKEEP,), jnp.int32),
    ],
)(_body)


def _thw_body(g_ref, o_ref):
    g = g_ref[...]
    half = g // 2
    col = lax.broadcasted_iota(jnp.int32, g.shape, 1)
    o_ref[...] = jnp.where(col == 0, g, half)


def _thw_compress(grid_thw):
    return pl.pallas_call(
        _thw_body,
        out_shape=jax.ShapeDtypeStruct((BATCH, 3), jnp.int32),
    )(grid_thw)


def kernel(pixel_values, grid_thw, input_ids, position_ids, attention_mask):
    perm = jnp.asarray(_PERM)
    pix, ids, pos, am = _sc_kernel(
        perm, pixel_values, input_ids, position_ids, attention_mask)
    thw = _thw_compress(grid_thw)
    return (pix, thw, ids, pos, am)


# speedup vs baseline: 5.0287x; 1.0216x over previous
import functools

import jax
import jax.numpy as jnp
from jax import lax
from jax.experimental import pallas as pl
from jax.experimental.pallas import tpu as pltpu
from jax.experimental.pallas import tpu_sc as plsc

IMAGE_TOKEN_ID = 151655
VIDEO_TOKEN_ID = 151656

BATCH = 4
SEQLEN = 4096
HIDDEN = 1280
T, H, W = 1, 64, 48
N_VIS = T * H * W
CNT = (H // 4) * (W // 4)
N_IMG = (H // 2) * (W // 2)
NKEEP = 100 + CNT + (SEQLEN - 100 - N_IMG)

L = 16
NCH = SEQLEN // L
NSEL = CNT // L
NOUT = NKEEP // L

NTILES = 32
NCHUNKS = BATCH * N_VIS
OROWS = 8
ROUNDS2 = (NCHUNKS // 4) // (NTILES * OROWS)
IDXSTRIDE = OROWS


def _make_perm():
    import numpy as np
    tbl = np.zeros(NTILES * 4 * 128, dtype=np.int32)
    for j in range(4):
        wid = np.arange(NTILES)[:, None, None]
        r = np.arange(ROUNDS2)[None, :, None]
        k = np.arange(IDXSTRIDE)[None, None, :]
        o = wid * (ROUNDS2 * OROWS) + r * OROWS + k
        i = o // 768
        P = (o // 384) % 2
        A = (o // 24) % 16
        u = (o // 12) % 2
        v = o % 12
        f, e = j // 2, j % 2
        perm = (i * N_VIS + (A * 4 + f * 2 + u) * W
                + (v * 4 + P * 2 + e)).reshape(NTILES, ROUNDS2 * IDXSTRIDE)
        sec = np.zeros((NTILES, 128), dtype=np.int32)
        sec[:, :ROUNDS2 * IDXSTRIDE] = perm
        view = tbl.reshape(NTILES, 4, 128)
        view[:, j, :] = sec
    return tbl


_PERM = _make_perm()


def _body(perm_hbm, pv_hbm, ids_hbm, pos_hbm, am_hbm,
          out_pix, out_ids, out_pos, out_am,
          buf0, buf1, ix_all, sem0, sem1, wsem0, wsem1,
          ids_v, am_v, pos_v0, pos_v1, pos_v2, mask_v, idx_v, stage_o):
    pos_v = (pos_v0, pos_v1, pos_v2)
    bufs = (buf0, buf1)
    sems = (sem0, sem1)
    wsems = (wsem0, wsem1)
    wid = lax.axis_index("s") * 2 + lax.axis_index("c")
    lane = lax.iota(jnp.int32, L)

    pltpu.sync_copy(perm_hbm.at[pl.ds(wid * 512, 512)], ix_all)

    def gather(r, b):
        hs = []
        for j in range(4):
            idx_j = ix_all.at[pl.ds(j * 128 + r * IDXSTRIDE, OROWS)]
            dst = bufs[b].at[:, pl.ds(j * HIDDEN, HIDDEN)]
            hs.append(pltpu.async_copy(pv_hbm.at[idx_j], dst, sems[b]))
        return hs

    def scatter(r, b):
        row0 = wid * (ROUNDS2 * OROWS) + r * OROWS
        return pltpu.async_copy(
            bufs[b], out_pix.at[pl.ds(row0, OROWS)], wsems[b])

    gh = [None, None]
    sh = [None, None]
    for r in range(ROUNDS2):
        b = r % 2
        if sh[b] is not None:
            sh[b].wait()
        gh[b] = gather(r, b)
        if r > 0:
            pb = (r - 1) % 2
            for h in gh[pb]:
                h.wait()
            sh[pb] = scatter(r - 1, pb)
    lb = (ROUNDS2 - 1) % 2
    for h in gh[lb]:
        h.wait()
    sh[lb] = scatter(ROUNDS2 - 1, lb)
    sh[1 - lb].wait()
    sh[lb].wait()

    @pl.when(wid < BATCH)
    def _():
        i = wid
        pltpu.sync_copy(ids_hbm.at[i], ids_v)
        pltpu.sync_copy(am_hbm.at[i], am_v)
        for k in range(3):
            pltpu.sync_copy(pos_hbm.at[k, i], pos_v[k])

        def span_body(c, carry):
            nmn, mx = carry
            t = lane + c * L
            v = ids_v[pl.ds(c * L, L)]
            med = (v == IMAGE_TOKEN_ID) | (v == VIDEO_TOKEN_ID)
            nmn = jnp.maximum(nmn, jnp.where(med, -t, -SEQLEN))
            mx = jnp.maximum(mx, jnp.where(med, t, -1))
            return nmn, mx

        nmn, mx = lax.fori_loop(
            0, NCH, span_body,
            (jnp.full((L,), -SEQLEN, jnp.int32),
             jnp.full((L,), -1, jnp.int32)))
        start = -jnp.max(nmn)
        end = jnp.max(mx)

        def mask_body(c, _):
            t = lane + c * L
            mask_v[pl.ds(c * L, L)] = ((t < start) | (t > end)).astype(jnp.int32)
            return 0

        lax.fori_loop(0, NCH, mask_body, 0)

        startf = start.astype(jnp.float32)
        endf = end.astype(jnp.float32)

        def sel_body(ck, _):
            kf = (lane + ck * L).astype(jnp.float32)
            h = kf / float(CNT - 1)
            sel = (startf * (1.0 - h) + endf * h).astype(jnp.int32)
            plsc.store_scatter(mask_v, [sel], jnp.ones((L,), jnp.int32))
            return 0

        lax.fori_loop(0, NSEL, sel_body, 0)

        def zero_body(j, _):
            idx_v[pl.ds(j * L, L)] = jnp.zeros((L,), jnp.int32)
            return 0

        lax.fori_loop(0, NOUT, zero_body, 0)

        def compact_body(c, base):
            t = lane + c * L
            m = mask_v[pl.ds(c * L, L)]
            a = am_v[pl.ds(c * L, L)]
            m = jnp.where(a != 0, m, 0)
            incl = plsc.cumsum(m)
            dest = base + incl - m
            ok = (m > 0) & (dest < NKEEP)
            plsc.store_scatter(idx_v, [dest], t, mask=ok)
            return base + jnp.sum(m)

        lax.fori_loop(0, NCH, compact_body, jnp.int32(0))

        def make_gather(src_v):
            def gather_body(j, _):
                iv = idx_v[pl.ds(j * L, L)]
                stage_o[pl.ds(j * L, L)] = plsc.load_gather(src_v, [iv])
                return 0
            return gather_body

        lax.fori_loop(0, NOUT, make_gather(ids_v), 0)
        pltpu.sync_copy(stage_o, out_ids.at[i])
        lax.fori_loop(0, NOUT, make_gather(am_v), 0)
        pltpu.sync_copy(stage_o, out_am.at[i])
        for k in range(3):
            lax.fori_loop(0, NOUT, make_gather(pos_v[k]), 0)
            pltpu.sync_copy(stage_o, out_pos.at[k, i])


_sc_kernel = functools.partial(
    pl.kernel,
    out_type=[
        jax.ShapeDtypeStruct((BATCH * N_VIS // 4, 4 * HIDDEN), jnp.float32),
        jax.ShapeDtypeStruct((BATCH, NKEEP), jnp.int32),
        jax.ShapeDtypeStruct((3, BATCH, NKEEP), jnp.int32),
        jax.ShapeDtypeStruct((BATCH, NKEEP), jnp.int32),
    ],
    mesh=plsc.VectorSubcoreMesh(core_axis_name="c", subcore_axis_name="s"),
    compiler_params=pltpu.CompilerParams(needs_layout_passes=False),
    scratch_types=[
        pltpu.VMEM((OROWS, 4 * HIDDEN), jnp.float32),
        pltpu.VMEM((OROWS, 4 * HIDDEN), jnp.float32),
        pltpu.VMEM((4 * 128,), jnp.int32),
        pltpu.SemaphoreType.DMA,
        pltpu.SemaphoreType.DMA,
        pltpu.SemaphoreType.DMA,
        pltpu.SemaphoreType.DMA,
        pltpu.VMEM((SEQLEN,), jnp.int32),
        pltpu.VMEM((SEQLEN,), jnp.int32),
        pltpu.VMEM((SEQLEN,), jnp.int32),
        pltpu.VMEM((SEQLEN,), jnp.int32),
        pltpu.VMEM((SEQLEN,), jnp.int32),
        pltpu.VMEM((SEQLEN,), jnp.int32),
        pltpu.VMEM((NKEEP,), jnp.int32),
        pltpu.VMEM((NKEEP,), jnp.int32),
    ],
)(_body)


def _thw_body(g_ref, o_ref):
    g = g_ref[...]
    half = g // 2
    col = lax.broadcasted_iota(jnp.int32, g.shape, 1)
    o_ref[...] = jnp.where(col == 0, g, half)


def _thw_compress(grid_thw):
    return pl.pallas_call(
        _thw_body,
        out_shape=jax.ShapeDtypeStruct((BATCH, 3), jnp.int32),
    )(grid_thw)


def kernel(pixel_values, grid_thw, input_ids, position_ids, attention_mask):
    perm = jnp.asarray(_PERM)
    pix, ids, pos, am = _sc_kernel(
        perm, pixel_values, input_ids, position_ids, attention_mask)
    thw = _thw_compress(grid_thw)
    return (pix, thw, ids, pos, am)
